# baseline (speedup 1.0000x reference)
import functools

import jax
import jax.numpy as jnp
from jax import lax
from jax.experimental import pallas as pl
from jax.experimental.pallas import tpu as pltpu
from jax.experimental.pallas import tpu_sc as plsc

D_MODEL = 1024
MAX_LEN = 8192
BATCH = 2
SEQ = 8192

NUM_CORES = 2
NUM_SUBCORES = 16
NW = NUM_CORES * NUM_SUBCORES
S_PER_W = SEQ // NW
K = 32
N_CHUNKS = S_PER_W // K
LANES = 16
VPR = D_MODEL // LANES

_mesh = plsc.VectorSubcoreMesh(core_axis_name="c", subcore_axis_name="s")


@functools.partial(
    pl.kernel,
    out_type=jax.ShapeDtypeStruct((BATCH, SEQ, D_MODEL), jnp.float32),
    mesh=_mesh,
    scratch_types=[
        pltpu.VMEM((S_PER_W,), jnp.int32),
        pltpu.VMEM((K, D_MODEL), jnp.float32),
        pltpu.VMEM((K, D_MODEL), jnp.float32),
        pltpu.SemaphoreType.DMA,
    ],
)
def _pe_add(inp_hbm, idx_hbm, pe_hbm, out_hbm, idx_v, pe_buf, in_buf, sem):
    wid = lax.axis_index("s") * NUM_CORES + lax.axis_index("c")
    base = wid * S_PER_W
    pltpu.sync_copy(idx_hbm.at[pl.ds(base, S_PER_W)], idx_v)

    def add_rows(r, _):
        for v in range(VPR):
            sl = pl.ds(v * LANES, LANES)
            in_buf[r, sl] = in_buf[r, sl] + pe_buf[r, sl]
        return 0

    for c in range(N_CHUNKS):
        s0 = base + c * K
        pltpu.async_copy(
            pe_hbm.at[idx_v.at[pl.ds(c * K, K)]], pe_buf, sem
        ).wait()
        for b in range(BATCH):
            pltpu.sync_copy(inp_hbm.at[b, pl.ds(s0, K)], in_buf)
            lax.fori_loop(0, K, add_rows, 0)
            pltpu.sync_copy(in_buf, out_hbm.at[b, pl.ds(s0, K)])


def kernel(input, indices, pe):
    return _pe_add(input, indices.astype(jnp.int32), pe)


# speedup vs baseline: 1.6980x; 1.6980x over previous
import functools

import jax
import jax.numpy as jnp
from jax import lax
from jax.experimental import pallas as pl
from jax.experimental.pallas import tpu as pltpu
from jax.experimental.pallas import tpu_sc as plsc

D_MODEL = 1024
MAX_LEN = 8192
BATCH = 2
SEQ = 8192

NUM_CORES = 2
NUM_SUBCORES = 16
NW = NUM_CORES * NUM_SUBCORES
S_PER_W = SEQ // NW
K = 8
N_CHUNKS = S_PER_W // K
DEPTH = 4
LANES = 16
VPR = D_MODEL // LANES

_mesh = plsc.VectorSubcoreMesh(core_axis_name="c", subcore_axis_name="s")


@functools.partial(
    pl.kernel,
    out_type=jax.ShapeDtypeStruct((BATCH, SEQ, D_MODEL), jnp.float32),
    mesh=_mesh,
    scratch_types=[
        pltpu.VMEM((S_PER_W,), jnp.int32),
        pltpu.VMEM((DEPTH, K, D_MODEL), jnp.float32),
        pltpu.VMEM((DEPTH * BATCH, K, D_MODEL), jnp.float32),
        pltpu.SemaphoreType.DMA((DEPTH,)),
        pltpu.SemaphoreType.DMA((DEPTH * BATCH,)),
        pltpu.SemaphoreType.DMA((DEPTH * BATCH,)),
    ],
)
def _pe_add(inp_hbm, idx_hbm, pe_hbm, out_hbm, idx_v, pe_buf, in_buf,
            sem_pe, sem_in, sem_out):
    wid = lax.axis_index("s") * NUM_CORES + lax.axis_index("c")
    base = wid * S_PER_W
    pltpu.sync_copy(idx_hbm.at[pl.ds(pl.multiple_of(base, 8), S_PER_W)],
                    idx_v)

    def issue_loads(c):
        slot = c % DEPTH
        off = pl.multiple_of(c * K, 8)
        s0 = pl.multiple_of(base + c * K, 8)
        pltpu.async_copy(pe_hbm.at[idx_v.at[pl.ds(off, K)]],
                         pe_buf.at[slot], sem_pe.at[slot])
        for b in range(BATCH):
            ib = slot * BATCH + b
            pltpu.async_copy(inp_hbm.at[b, pl.ds(s0, K)],
                             in_buf.at[ib], sem_in.at[ib])

    def wait_loads(slot):
        pltpu.make_async_copy(pe_hbm.at[pl.ds(0, K)], pe_buf.at[slot],
                              sem_pe.at[slot]).wait()
        for b in range(BATCH):
            ib = slot * BATCH + b
            pltpu.make_async_copy(inp_hbm.at[b, pl.ds(0, K)],
                                  in_buf.at[ib], sem_in.at[ib]).wait()

    def issue_stores(c):
        slot = c % DEPTH
        s0 = pl.multiple_of(base + c * K, 8)
        for b in range(BATCH):
            ib = slot * BATCH + b
            pltpu.async_copy(in_buf.at[ib], out_hbm.at[b, pl.ds(s0, K)],
                             sem_out.at[ib])

    def wait_stores(slot):
        for b in range(BATCH):
            ib = slot * BATCH + b
            pltpu.make_async_copy(in_buf.at[ib],
                                  out_hbm.at[b, pl.ds(0, K)],
                                  sem_out.at[ib]).wait()

    issue_loads(0)
    issue_loads(1)

    def chunk_body(c, _):
        slot = c % DEPTH

        @pl.when(c + 2 < N_CHUNKS)
        def _():
            @pl.when(c >= 2)
            def _():
                wait_stores((c + 2) % DEPTH)
            issue_loads(c + 2)

        wait_loads(slot)
        i0 = slot * BATCH
        i1 = i0 + 1

        def add_rows(r, _):
            for v in range(VPR):
                sl = pl.ds(v * LANES, LANES)
                pv = pe_buf[slot, r, sl]
                plsc.addupdate(in_buf.at[i0, r, sl], pv)
                plsc.addupdate(in_buf.at[i1, r, sl], pv)
            return 0

        lax.fori_loop(0, K, add_rows, 0)
        issue_stores(c)
        return 0

    lax.fori_loop(0, N_CHUNKS, chunk_body, 0)

    for slot in range(DEPTH):
        wait_stores(slot)


def kernel(input, indices, pe):
    return _pe_add(input, indices.astype(jnp.int32), pe)
